# baseline (speedup 1.0000x reference)
import jax
import jax.numpy as jnp
from jax import lax
from jax.experimental import pallas as pl
from jax.experimental.pallas import tpu as pltpu
from jax.experimental.pallas import tpu_sc as plsc

NC = 2
NS = 16
NW = NC * NS
K = 8


def kernel(inputs, embedding):
    batch, hist = inputs.shape
    features = embedding.shape[1]
    assert batch % NW == 0
    rows_per_w = batch // NW
    assert rows_per_w % K == 0
    groups = rows_per_w // K
    assert groups % 2 == 0
    assert hist <= 128
    hp = (hist + 7) // 8 * 8

    def body(table_hbm, idx_hbm, out_hbm, idx_v, rows0, rows1,
             sg0, sg1, so0, so1):
        rows = (rows0, rows1)
        sem_g = (sg0, sg1)
        sem_o = (so0, so1)
        wid = lax.axis_index("s") * NC + lax.axis_index("c")
        base = wid * rows_per_w
        pltpu.sync_copy(
            idx_hbm.at[pl.ds(base, rows_per_w), pl.ds(0, hp)], idx_v)

        def fire_gathers(g, buf, sem):
            for j in range(K):
                pltpu.async_copy(table_hbm.at[idx_v.at[g * K + j]],
                                 buf.at[j], sem)

        def wait_gathers(g, buf, sem):
            for j in range(K):
                pltpu.make_async_copy(table_hbm.at[idx_v.at[g * K + j]],
                                      buf.at[j], sem).wait()

        def fire_out(g, buf, sem):
            pltpu.async_copy(buf, out_hbm.at[pl.ds(base + g * K, K)], sem)

        def drain_out(buf, sem):
            pltpu.make_async_copy(buf, out_hbm.at[pl.ds(base, K)], sem).wait()

        fire_gathers(0, rows[0], sem_g[0])

        @pl.loop(0, groups, step=2)
        def _(g0):
            for b in range(2):
                g = g0 + b
                nb = 1 - b

                @pl.when(g + 1 < groups)
                def _():
                    @pl.when(g >= 1)
                    def _():
                        drain_out(rows[nb], sem_o[nb])
                    fire_gathers(g + 1, rows[nb], sem_g[nb])

                wait_gathers(g, rows[b], sem_g[b])
                fire_out(g, rows[b], sem_o[b])

        drain_out(rows[0], sem_o[0])
        drain_out(rows[1], sem_o[1])

    padded = pl.kernel(
        body,
        out_type=jax.ShapeDtypeStruct((batch, hp, features), jnp.float32),
        mesh=plsc.VectorSubcoreMesh(core_axis_name="c", subcore_axis_name="s"),
        scratch_types=[
            pltpu.VMEM((rows_per_w, hp), jnp.int32),
            pltpu.VMEM((K, hp, features), jnp.float32),
            pltpu.VMEM((K, hp, features), jnp.float32),
            pltpu.SemaphoreType.DMA,
            pltpu.SemaphoreType.DMA,
            pltpu.SemaphoreType.DMA,
            pltpu.SemaphoreType.DMA,
        ],
        compiler_params=pltpu.CompilerParams(use_tc_tiling_on_sc=False),
    )(embedding, jnp.pad(inputs, ((0, 0), (0, 128 - hist))))
    return padded[:, :hist, :]


# speedup vs baseline: 3.0191x; 3.0191x over previous
import jax
import jax.numpy as jnp
from jax import lax
from jax.experimental import pallas as pl
from jax.experimental.pallas import tpu as pltpu
from jax.experimental.pallas import tpu_sc as plsc

NC = 2
NS = 16
NW = NC * NS
K = 4
LANES = 16


def kernel(inputs, embedding):
    batch, hist = inputs.shape
    features = embedding.shape[1]
    assert batch % NW == 0
    rows_per_w = batch // NW
    assert rows_per_w % (2 * K) == 0
    pairs = rows_per_w // (2 * K)
    assert features % LANES == 0
    fblocks = features // LANES

    def body(table_hbm, idx_hbm, out_hbm, idx0, idx1, rows0, rows1, sel,
             si0, si1, sg0, sg1, so):
        idxs = (idx0, idx1)
        rows = (rows0, rows1)
        sem_i = (si0, si1)
        sem_g = (sg0, sg1)
        wid = lax.axis_index("s") * NC + lax.axis_index("c")
        base = wid * rows_per_w

        def fire_idx(h, buf, sem):
            pltpu.async_copy(idx_hbm.at[pl.ds(base + h * K, K)], buf, sem)

        def wait_idx(h, buf, sem):
            pltpu.make_async_copy(idx_hbm.at[pl.ds(base + h * K, K)],
                                  buf, sem).wait()

        def fire_gathers(buf, idx_v, sem):
            for j in range(K):
                pltpu.async_copy(table_hbm.at[idx_v.at[j]], buf.at[j], sem)

        def wait_gathers(buf, idx_v, sem):
            for j in range(K):
                pltpu.make_async_copy(table_hbm.at[idx_v.at[j]],
                                      buf.at[j], sem).wait()

        def compact(buf, half):
            for j in range(K):
                @pl.loop(0, hist)
                def _(r):
                    for c in range(fblocks):
                        sel[half * K + j, r, pl.ds(c * LANES, LANES)] = (
                            buf[j, r, pl.ds(c * LANES, LANES)])

        def fire_out(p):
            pltpu.async_copy(sel, out_hbm.at[pl.ds(base + p * 2 * K, 2 * K)],
                             so)

        def drain_out():
            pltpu.make_async_copy(sel, out_hbm.at[pl.ds(base, 2 * K)],
                                  so).wait()

        fire_idx(0, idxs[0], sem_i[0])
        fire_idx(1, idxs[1], sem_i[1])
        wait_idx(0, idxs[0], sem_i[0])
        fire_gathers(rows[0], idxs[0], sem_g[0])

        @pl.loop(0, pairs)
        def _(p):
            for b in range(2):
                h = p * 2 + b
                nb = 1 - b
                @pl.when(h + 1 < 2 * pairs)
                def _():
                    wait_idx(h + 1, idxs[nb], sem_i[nb])
                    fire_gathers(rows[nb], idxs[nb], sem_g[nb])
                wait_gathers(rows[b], idxs[b], sem_g[b])
                @pl.when(h + 2 < 2 * pairs)
                def _():
                    fire_idx(h + 2, idxs[b], sem_i[b])
                if b == 0:
                    @pl.when(p > 0)
                    def _():
                        drain_out()
                compact(rows[b], b)
            fire_out(p)

        drain_out()

    padded_table = jnp.pad(embedding, ((0, 0), (0, 128 - features)))
    return pl.kernel(
        body,
        out_type=jax.ShapeDtypeStruct((batch, hist, features), jnp.float32),
        mesh=plsc.VectorSubcoreMesh(core_axis_name="c", subcore_axis_name="s"),
        scratch_types=[
            pltpu.VMEM((K, hist), jnp.int32),
            pltpu.VMEM((K, hist), jnp.int32),
            pltpu.VMEM((K, hist, 128), jnp.float32),
            pltpu.VMEM((K, hist, 128), jnp.float32),
            pltpu.VMEM((2 * K, hist, features), jnp.float32),
            pltpu.SemaphoreType.DMA,
            pltpu.SemaphoreType.DMA,
            pltpu.SemaphoreType.DMA,
            pltpu.SemaphoreType.DMA,
            pltpu.SemaphoreType.DMA,
        ],
        compiler_params=pltpu.CompilerParams(use_tc_tiling_on_sc=True),
    )(padded_table, inputs)
